# baseline (speedup 1.0000x reference)
import jax
import jax.numpy as jnp
from jax.experimental import pallas as pl

_N = 256
_F = 256
_H = 64
_CHUNK = 64


def _gat_layer(x, Wl, Wr, a2d, b2d):
    xl = jnp.dot(x, Wl, preferred_element_type=jnp.float32)
    rows = []
    xr = jnp.dot(x, Wr, preferred_element_type=jnp.float32)
    for i0 in range(0, _N, _CHUNK):
        u = xr[i0:i0 + _CHUNK][:, None, :] + xl[None, :, :]
        s = jnp.where(u >= 0, u, 0.2 * u) * a2d[None, :, :]
        rows.append(jnp.sum(s, axis=-1))
    scores = jnp.concatenate(rows, axis=0)
    m = jnp.max(scores, axis=1, keepdims=True)
    p = jnp.exp(scores - m)
    alpha = p / jnp.sum(p, axis=1, keepdims=True)
    out = jnp.dot(alpha, xl, preferred_element_type=jnp.float32) + b2d
    return jnp.maximum(out, 0.0)


def _net_kernel(x_ref, w1l_ref, w1r_ref, a1_ref, b1_ref,
                w2l_ref, w2r_ref, a2_ref, b2_ref, out_ref):
    x = x_ref[0]
    h1 = _gat_layer(x, w1l_ref[0], w1r_ref[0], a1_ref[0], b1_ref[0])
    h2 = _gat_layer(h1, w2l_ref[0], w2r_ref[0], a2_ref[0], b2_ref[0])
    out_ref[0, 0] = jnp.mean(h2, axis=0, keepdims=True)


def kernel(features, pi1_Wl, pi1_Wr, pi1_a, pi1_b, pi2_Wl, pi2_Wr, pi2_a, pi2_b,
           vf1_Wl, vf1_Wr, vf1_a, vf1_b, vf2_Wl, vf2_Wr, vf2_a, vf2_b):
    B = features.shape[0]
    w1l = jnp.stack([pi1_Wl, vf1_Wl])
    w1r = jnp.stack([pi1_Wr, vf1_Wr])
    w2l = jnp.stack([pi2_Wl, vf2_Wl])
    w2r = jnp.stack([pi2_Wr, vf2_Wr])
    a1 = jnp.stack([pi1_a, vf1_a])[:, None, :]
    a2 = jnp.stack([pi2_a, vf2_a])[:, None, :]
    b1 = jnp.stack([pi1_b, vf1_b])[:, None, :]
    b2 = jnp.stack([pi2_b, vf2_b])[:, None, :]

    out = pl.pallas_call(
        _net_kernel,
        grid=(2, B),
        in_specs=[
            pl.BlockSpec((1, _N, _F), lambda r, b: (b, 0, 0)),
            pl.BlockSpec((1, _F, _H), lambda r, b: (r, 0, 0)),
            pl.BlockSpec((1, _F, _H), lambda r, b: (r, 0, 0)),
            pl.BlockSpec((1, 1, _H), lambda r, b: (r, 0, 0)),
            pl.BlockSpec((1, 1, _H), lambda r, b: (r, 0, 0)),
            pl.BlockSpec((1, _H, _H), lambda r, b: (r, 0, 0)),
            pl.BlockSpec((1, _H, _H), lambda r, b: (r, 0, 0)),
            pl.BlockSpec((1, 1, _H), lambda r, b: (r, 0, 0)),
            pl.BlockSpec((1, 1, _H), lambda r, b: (r, 0, 0)),
        ],
        out_specs=pl.BlockSpec((1, 1, 1, _H), lambda r, b: (r, b, 0, 0)),
        out_shape=jax.ShapeDtypeStruct((2, B, 1, _H), jnp.float32),
    )(features, w1l, w1r, a1, b1, w2l, w2r, a2, b2)

    return out[0, :, 0, :], out[1, :, 0, :]


# speedup vs baseline: 2.6949x; 2.6949x over previous
import jax
import jax.numpy as jnp
from jax.experimental import pallas as pl
from jax.experimental.pallas import tpu as pltpu

_N = 256
_F = 256
_H = 64


def _gat_layer(x, Wl, Wr, a_row, a_col, coef_ref, b2d):
    xr = jnp.dot(x, Wr, preferred_element_type=jnp.float32)
    xlT = jax.lax.dot_general(Wl, x, (((0,), (1,)), ((), ())),
                              preferred_element_type=jnp.float32)
    p = xr * jnp.abs(a_row)
    qT = xlT * jnp.abs(a_col)
    ar = jnp.dot(xr, a_col, preferred_element_type=jnp.float32)
    al = jnp.dot(a_row, xlT, preferred_element_type=jnp.float32)
    scores = 0.2 * (ar + al)
    for d in range(_H):
        w = p[:, d:d + 1] + qT[d:d + 1, :]
        scores = scores + coef_ref[0, 0, d] * jnp.maximum(w, 0.0)
    m = jnp.max(scores, axis=1, keepdims=True)
    e = jnp.exp(scores - m)
    alpha = e * (1.0 / jnp.sum(e, axis=1, keepdims=True))
    out = jax.lax.dot_general(alpha, xlT, (((1,), (1,)), ((), ())),
                              preferred_element_type=jnp.float32) + b2d
    return jnp.maximum(out, 0.0)


def _net_kernel(x_ref, w1l_ref, w1r_ref, a1r_ref, a1c_ref, c1_ref, b1_ref,
                w2l_ref, w2r_ref, a2r_ref, a2c_ref, c2_ref, b2_ref, out_ref):
    x = x_ref[0]
    h1 = _gat_layer(x, w1l_ref[0], w1r_ref[0], a1r_ref[0], a1c_ref[0],
                    c1_ref, b1_ref[0])
    h2 = _gat_layer(h1, w2l_ref[0], w2r_ref[0], a2r_ref[0], a2c_ref[0],
                    c2_ref, b2_ref[0])
    out_ref[0, 0] = jnp.mean(h2, axis=0, keepdims=True)


def kernel(features, pi1_Wl, pi1_Wr, pi1_a, pi1_b, pi2_Wl, pi2_Wr, pi2_a, pi2_b,
           vf1_Wl, vf1_Wr, vf1_a, vf1_b, vf2_Wl, vf2_Wr, vf2_a, vf2_b):
    B = features.shape[0]
    w1l = jnp.stack([pi1_Wl, vf1_Wl])
    w1r = jnp.stack([pi1_Wr, vf1_Wr])
    w2l = jnp.stack([pi2_Wl, vf2_Wl])
    w2r = jnp.stack([pi2_Wr, vf2_Wr])
    a1 = jnp.stack([pi1_a, vf1_a])
    a2 = jnp.stack([pi2_a, vf2_a])
    c1 = 0.8 * jnp.where(a1 >= 0, 1.0, -1.0)[:, None, :]
    c2 = 0.8 * jnp.where(a2 >= 0, 1.0, -1.0)[:, None, :]
    a1r = a1[:, None, :]
    a2r = a2[:, None, :]
    a1c = a1[:, :, None]
    a2c = a2[:, :, None]
    b1 = jnp.stack([pi1_b, vf1_b])[:, None, :]
    b2 = jnp.stack([pi2_b, vf2_b])[:, None, :]

    wspec = lambda s: pl.BlockSpec((1,) + s, lambda r, b: (r, 0, 0))
    sspec = pl.BlockSpec((1, 1, _H), lambda r, b: (r, 0, 0),
                         memory_space=pltpu.SMEM)

    out = pl.pallas_call(
        _net_kernel,
        grid=(2, B),
        in_specs=[
            pl.BlockSpec((1, _N, _F), lambda r, b: (b, 0, 0)),
            wspec((_F, _H)), wspec((_F, _H)),
            wspec((1, _H)), wspec((_H, 1)), sspec, wspec((1, _H)),
            wspec((_H, _H)), wspec((_H, _H)),
            wspec((1, _H)), wspec((_H, 1)), sspec, wspec((1, _H)),
        ],
        out_specs=pl.BlockSpec((1, 1, 1, _H), lambda r, b: (r, b, 0, 0)),
        out_shape=jax.ShapeDtypeStruct((2, B, 1, _H), jnp.float32),
    )(features, w1l, w1r, a1r, a1c, c1, b1, w2l, w2r, a2r, a2c, c2, b2)

    return out[0, :, 0, :], out[1, :, 0, :]


# speedup vs baseline: 2.6951x; 1.0001x over previous
import jax
import jax.numpy as jnp
from jax.experimental import pallas as pl
from jax.experimental.pallas import tpu as pltpu

_N = 256
_F = 256
_H = 64
_BI = 256


def _gat_layer(x, Wl, Wr, a_row, a_col, coef_ref, b2d):
    xr = jnp.dot(x, Wr, preferred_element_type=jnp.float32)
    xlT = jax.lax.dot_general(Wl, x, (((0,), (1,)), ((), ())),
                              preferred_element_type=jnp.float32)
    p = xr * jnp.abs(a_row)
    qT = xlT * jnp.abs(a_col)
    ar = jnp.dot(xr, a_col, preferred_element_type=jnp.float32)
    al = jnp.dot(a_row, xlT, preferred_element_type=jnp.float32)
    blocks = []
    for i0 in range(0, _N, _BI):
        pb = p[i0:i0 + _BI, :]
        acc = 0.2 * (ar[i0:i0 + _BI, :] + al)
        for d in range(_H):
            w = pb[:, d:d + 1] + qT[d:d + 1, :]
            acc = acc + coef_ref[0, 0, d] * jnp.maximum(w, 0.0)
        blocks.append(acc)
    scores = jnp.concatenate(blocks, axis=0)
    m = jnp.max(scores, axis=1, keepdims=True)
    e = jnp.exp(scores - m)
    alpha = e * (1.0 / jnp.sum(e, axis=1, keepdims=True))
    out = jax.lax.dot_general(alpha, xlT, (((1,), (1,)), ((), ())),
                              preferred_element_type=jnp.float32) + b2d
    return jnp.maximum(out, 0.0)


def _net_kernel(x_ref, w1l_ref, w1r_ref, a1r_ref, a1c_ref, c1_ref, b1_ref,
                w2l_ref, w2r_ref, a2r_ref, a2c_ref, c2_ref, b2_ref, out_ref):
    x = x_ref[0]
    h1 = _gat_layer(x, w1l_ref[0], w1r_ref[0], a1r_ref[0], a1c_ref[0],
                    c1_ref, b1_ref[0])
    h2 = _gat_layer(h1, w2l_ref[0], w2r_ref[0], a2r_ref[0], a2c_ref[0],
                    c2_ref, b2_ref[0])
    out_ref[0, 0] = jnp.mean(h2, axis=0, keepdims=True)


def kernel(features, pi1_Wl, pi1_Wr, pi1_a, pi1_b, pi2_Wl, pi2_Wr, pi2_a, pi2_b,
           vf1_Wl, vf1_Wr, vf1_a, vf1_b, vf2_Wl, vf2_Wr, vf2_a, vf2_b):
    B = features.shape[0]
    w1l = jnp.stack([pi1_Wl, vf1_Wl])
    w1r = jnp.stack([pi1_Wr, vf1_Wr])
    w2l = jnp.stack([pi2_Wl, vf2_Wl])
    w2r = jnp.stack([pi2_Wr, vf2_Wr])
    a1 = jnp.stack([pi1_a, vf1_a])
    a2 = jnp.stack([pi2_a, vf2_a])
    c1 = 0.8 * jnp.where(a1 >= 0, 1.0, -1.0)[:, None, :]
    c2 = 0.8 * jnp.where(a2 >= 0, 1.0, -1.0)[:, None, :]
    a1r = a1[:, None, :]
    a2r = a2[:, None, :]
    a1c = a1[:, :, None]
    a2c = a2[:, :, None]
    b1 = jnp.stack([pi1_b, vf1_b])[:, None, :]
    b2 = jnp.stack([pi2_b, vf2_b])[:, None, :]

    wspec = lambda s: pl.BlockSpec((1,) + s, lambda r, b: (r, 0, 0))
    sspec = pl.BlockSpec((1, 1, _H), lambda r, b: (r, 0, 0),
                         memory_space=pltpu.SMEM)

    out = pl.pallas_call(
        _net_kernel,
        grid=(2, B),
        in_specs=[
            pl.BlockSpec((1, _N, _F), lambda r, b: (b, 0, 0)),
            wspec((_F, _H)), wspec((_F, _H)),
            wspec((1, _H)), wspec((_H, 1)), sspec, wspec((1, _H)),
            wspec((_H, _H)), wspec((_H, _H)),
            wspec((1, _H)), wspec((_H, 1)), sspec, wspec((1, _H)),
        ],
        out_specs=pl.BlockSpec((1, 1, 1, _H), lambda r, b: (r, b, 0, 0)),
        out_shape=jax.ShapeDtypeStruct((2, B, 1, _H), jnp.float32),
    )(features, w1l, w1r, a1r, a1c, c1, b1, w2l, w2r, a2r, a2c, c2, b2)

    return out[0, :, 0, :], out[1, :, 0, :]


# speedup vs baseline: 2.7731x; 1.0289x over previous
import jax
import jax.numpy as jnp
from jax.experimental import pallas as pl
from jax.experimental.pallas import tpu as pltpu

_N = 256
_F = 256
_H = 64
_BI = 256
_GD = 16


def _gat_layer(x, Wl, Wr, a_row, a_col, coef_ref, b2d):
    xr = jnp.dot(x, Wr, preferred_element_type=jnp.float32)
    xlT = jax.lax.dot_general(Wl, x, (((0,), (1,)), ((), ())),
                              preferred_element_type=jnp.float32)
    p = xr * jnp.abs(a_row)
    qT = xlT * jnp.abs(a_col)
    ar = jnp.dot(xr, a_col, preferred_element_type=jnp.float32)
    al = jnp.dot(a_row, xlT, preferred_element_type=jnp.float32)
    blocks = []
    for i0 in range(0, _N, _BI):
        pb = p[i0:i0 + _BI, :]
        acc = 0.2 * (ar[i0:i0 + _BI, :] + al)
        for d0 in range(0, _H, _GD):
            terms = [coef_ref[0, 0, d] *
                     jnp.maximum(pb[:, d:d + 1] + qT[d:d + 1, :], 0.0)
                     for d in range(d0, d0 + _GD)]
            while len(terms) > 1:
                terms = [terms[k] + terms[k + 1]
                         for k in range(0, len(terms) - 1, 2)] + \
                        (terms[-1:] if len(terms) % 2 else [])
            acc = acc + terms[0]
        blocks.append(acc)
    scores = jnp.concatenate(blocks, axis=0)
    m = jnp.max(scores, axis=1, keepdims=True)
    e = jnp.exp(scores - m)
    alpha = e * (1.0 / jnp.sum(e, axis=1, keepdims=True))
    out = jax.lax.dot_general(alpha, xlT, (((1,), (1,)), ((), ())),
                              preferred_element_type=jnp.float32) + b2d
    return jnp.maximum(out, 0.0)


def _net_kernel(x_ref, w1l_ref, w1r_ref, a1r_ref, a1c_ref, c1_ref, b1_ref,
                w2l_ref, w2r_ref, a2r_ref, a2c_ref, c2_ref, b2_ref, out_ref):
    x = x_ref[0]
    h1 = _gat_layer(x, w1l_ref[0], w1r_ref[0], a1r_ref[0], a1c_ref[0],
                    c1_ref, b1_ref[0])
    h2 = _gat_layer(h1, w2l_ref[0], w2r_ref[0], a2r_ref[0], a2c_ref[0],
                    c2_ref, b2_ref[0])
    out_ref[0, 0] = jnp.mean(h2, axis=0, keepdims=True)


def kernel(features, pi1_Wl, pi1_Wr, pi1_a, pi1_b, pi2_Wl, pi2_Wr, pi2_a, pi2_b,
           vf1_Wl, vf1_Wr, vf1_a, vf1_b, vf2_Wl, vf2_Wr, vf2_a, vf2_b):
    B = features.shape[0]
    w1l = jnp.stack([pi1_Wl, vf1_Wl])
    w1r = jnp.stack([pi1_Wr, vf1_Wr])
    w2l = jnp.stack([pi2_Wl, vf2_Wl])
    w2r = jnp.stack([pi2_Wr, vf2_Wr])
    a1 = jnp.stack([pi1_a, vf1_a])
    a2 = jnp.stack([pi2_a, vf2_a])
    c1 = 0.8 * jnp.where(a1 >= 0, 1.0, -1.0)[:, None, :]
    c2 = 0.8 * jnp.where(a2 >= 0, 1.0, -1.0)[:, None, :]
    a1r = a1[:, None, :]
    a2r = a2[:, None, :]
    a1c = a1[:, :, None]
    a2c = a2[:, :, None]
    b1 = jnp.stack([pi1_b, vf1_b])[:, None, :]
    b2 = jnp.stack([pi2_b, vf2_b])[:, None, :]

    wspec = lambda s: pl.BlockSpec((1,) + s, lambda r, b: (r, 0, 0))
    sspec = pl.BlockSpec((1, 1, _H), lambda r, b: (r, 0, 0),
                         memory_space=pltpu.SMEM)

    out = pl.pallas_call(
        _net_kernel,
        grid=(2, B),
        in_specs=[
            pl.BlockSpec((1, _N, _F), lambda r, b: (b, 0, 0)),
            wspec((_F, _H)), wspec((_F, _H)),
            wspec((1, _H)), wspec((_H, 1)), sspec, wspec((1, _H)),
            wspec((_H, _H)), wspec((_H, _H)),
            wspec((1, _H)), wspec((_H, 1)), sspec, wspec((1, _H)),
        ],
        out_specs=pl.BlockSpec((1, 1, 1, _H), lambda r, b: (r, b, 0, 0)),
        out_shape=jax.ShapeDtypeStruct((2, B, 1, _H), jnp.float32),
    )(features, w1l, w1r, a1r, a1c, c1, b1, w2l, w2r, a2r, a2c, c2, b2)

    return out[0, :, 0, :], out[1, :, 0, :]


# speedup vs baseline: 2.9090x; 1.0490x over previous
import jax
import jax.numpy as jnp
from jax.experimental import pallas as pl
from jax.experimental.pallas import tpu as pltpu

_N = 256
_F = 256
_H = 64
_GD = 16


def _scores_pair(p_cat, qT_cat, base_p, base_v, coef_ref):
    accp = base_p
    accv = base_v
    for d0 in range(0, _H, _GD):
        terms_p = []
        terms_v = []
        for d in range(d0, d0 + _GD):
            terms_p.append(coef_ref[0, d] *
                           jnp.maximum(p_cat[:, d:d + 1] +
                                       qT_cat[d:d + 1, :], 0.0))
            terms_v.append(coef_ref[0, _H + d] *
                           jnp.maximum(p_cat[:, _H + d:_H + d + 1] +
                                       qT_cat[_H + d:_H + d + 1, :], 0.0))
        while len(terms_p) > 1:
            terms_p = [terms_p[k] + terms_p[k + 1]
                       for k in range(0, len(terms_p) - 1, 2)]
            terms_v = [terms_v[k] + terms_v[k + 1]
                       for k in range(0, len(terms_v) - 1, 2)]
        accp = accp + terms_p[0]
        accv = accv + terms_v[0]
    return accp, accv


def _softmax_rows(s):
    m = jnp.max(s, axis=1, keepdims=True)
    e = jnp.exp(s - m)
    return e * (1.0 / jnp.sum(e, axis=1, keepdims=True))


def _dual_layer(x_cat, wl, wr, ar_cat, ac_cat, ar_blk, ac_blk, coef_ref,
                b_cat):
    xr_cat = jnp.dot(x_cat, wr, preferred_element_type=jnp.float32)
    xlT_cat = jax.lax.dot_general(wl, x_cat, (((0,), (1,)), ((), ())),
                                  preferred_element_type=jnp.float32)
    p_cat = xr_cat * jnp.abs(ar_cat)
    qT_cat = xlT_cat * jnp.abs(ac_cat)
    ar2 = jnp.dot(xr_cat, ac_blk, preferred_element_type=jnp.float32)
    al2 = jnp.dot(ar_blk, xlT_cat, preferred_element_type=jnp.float32)
    sp, sv = _scores_pair(p_cat, qT_cat,
                          0.2 * (ar2[:, 0:1] + al2[0:1, :]),
                          0.2 * (ar2[:, 1:2] + al2[1:2, :]), coef_ref)
    alp = _softmax_rows(sp)
    alv = _softmax_rows(sv)
    op = jax.lax.dot_general(alp, xlT_cat[:_H, :], (((1,), (1,)), ((), ())),
                             preferred_element_type=jnp.float32)
    ov = jax.lax.dot_general(alv, xlT_cat[_H:, :], (((1,), (1,)), ((), ())),
                             preferred_element_type=jnp.float32)
    h_cat = jnp.concatenate([op, ov], axis=1) + b_cat
    return jnp.maximum(h_cat, 0.0)


def _net_kernel(x_ref,
                wl1_ref, wr1_ref, ar1_ref, ac1_ref, arb1_ref, acb1_ref,
                c1_ref, b1_ref,
                wl2_ref, wr2_ref, ar2_ref, ac2_ref, arb2_ref, acb2_ref,
                c2_ref, b2_ref, out_ref):
    x = x_ref[0]
    h1 = _dual_layer(x, wl1_ref[...], wr1_ref[...], ar1_ref[...],
                     ac1_ref[...], arb1_ref[...], acb1_ref[...],
                     c1_ref, b1_ref[...])
    h2 = _dual_layer(h1, wl2_ref[...], wr2_ref[...], ar2_ref[...],
                     ac2_ref[...], arb2_ref[...], acb2_ref[...],
                     c2_ref, b2_ref[...])
    mean = jnp.mean(h2, axis=0, keepdims=True)
    out_ref[0, 0] = mean[:, :_H]
    out_ref[0, 1] = mean[:, _H:]


def kernel(features, pi1_Wl, pi1_Wr, pi1_a, pi1_b, pi2_Wl, pi2_Wr, pi2_a, pi2_b,
           vf1_Wl, vf1_Wr, vf1_a, vf1_b, vf2_Wl, vf2_Wr, vf2_a, vf2_b):
    B = features.shape[0]
    H = _H
    z = jnp.zeros((H, H), jnp.float32)
    zv = jnp.zeros((H,), jnp.float32)

    wl1 = jnp.concatenate([pi1_Wl, vf1_Wl], axis=1)
    wr1 = jnp.concatenate([pi1_Wr, vf1_Wr], axis=1)
    wl2 = jnp.block([[pi2_Wl, z], [z, vf2_Wl]])
    wr2 = jnp.block([[pi2_Wr, z], [z, vf2_Wr]])

    a1c = jnp.concatenate([pi1_a, vf1_a])
    a2c = jnp.concatenate([pi2_a, vf2_a])
    ar1 = a1c[None, :]
    ar2 = a2c[None, :]
    ac1 = a1c[:, None]
    ac2 = a2c[:, None]
    arb1 = jnp.stack([jnp.concatenate([pi1_a, zv]),
                      jnp.concatenate([zv, vf1_a])])
    arb2 = jnp.stack([jnp.concatenate([pi2_a, zv]),
                      jnp.concatenate([zv, vf2_a])])
    acb1 = arb1.T
    acb2 = arb2.T
    c1 = (0.8 * jnp.where(a1c >= 0, 1.0, -1.0))[None, :]
    c2 = (0.8 * jnp.where(a2c >= 0, 1.0, -1.0))[None, :]
    b1 = jnp.concatenate([pi1_b, vf1_b])[None, :]
    b2 = jnp.concatenate([pi2_b, vf2_b])[None, :]

    full = lambda arr: pl.BlockSpec(arr.shape, lambda b: (0,) * arr.ndim)
    smem = lambda arr: pl.BlockSpec(arr.shape, lambda b: (0,) * arr.ndim,
                                    memory_space=pltpu.SMEM)

    out = pl.pallas_call(
        _net_kernel,
        grid=(B,),
        in_specs=[
            pl.BlockSpec((1, _N, _F), lambda b: (b, 0, 0)),
            full(wl1), full(wr1), full(ar1), full(ac1), full(arb1),
            full(acb1), smem(c1), full(b1),
            full(wl2), full(wr2), full(ar2), full(ac2), full(arb2),
            full(acb2), smem(c2), full(b2),
        ],
        out_specs=pl.BlockSpec((1, 2, 1, H), lambda b: (b, 0, 0, 0)),
        out_shape=jax.ShapeDtypeStruct((B, 2, 1, H), jnp.float32),
    )(features, wl1, wr1, ar1, ac1, arb1, acb1, c1, b1,
      wl2, wr2, ar2, ac2, arb2, acb2, c2, b2)

    return out[:, 0, 0, :], out[:, 1, 0, :]


# speedup vs baseline: 2.9215x; 1.0043x over previous
import jax
import jax.numpy as jnp
from jax.experimental import pallas as pl
from jax.experimental.pallas import tpu as pltpu

_N = 256
_F = 256
_H = 64
_GD = 16


def _scores_pair(p_cat, qT_cat, base_p, base_v, coef_ref):
    accp = base_p
    accv = base_v
    for d0 in range(0, _H, _GD):
        terms_p = []
        terms_v = []
        for d in range(d0, d0 + _GD):
            terms_p.append(coef_ref[0, d] *
                           jnp.maximum(p_cat[:, d:d + 1] +
                                       qT_cat[d:d + 1, :], 0.0))
            terms_v.append(coef_ref[0, _H + d] *
                           jnp.maximum(p_cat[:, _H + d:_H + d + 1] +
                                       qT_cat[_H + d:_H + d + 1, :], 0.0))
        while len(terms_p) > 1:
            terms_p = [terms_p[k] + terms_p[k + 1]
                       for k in range(0, len(terms_p) - 1, 2)]
            terms_v = [terms_v[k] + terms_v[k + 1]
                       for k in range(0, len(terms_v) - 1, 2)]
        accp = accp + terms_p[0]
        accv = accv + terms_v[0]
    return accp, accv


def _softmax_unnorm(s):
    m = jnp.max(s, axis=1, keepdims=True)
    e = jnp.exp(s - m)
    return e, 1.0 / jnp.sum(e, axis=1, keepdims=True)


def _dual_layer(x_cat, wl, wr, ar_cat, ac_cat, ar_blk, ac_blk, coef_ref,
                b_cat):
    xr_cat = jnp.dot(x_cat, wr, preferred_element_type=jnp.float32)
    xlT_cat = jax.lax.dot_general(wl, x_cat, (((0,), (1,)), ((), ())),
                                  preferred_element_type=jnp.float32)
    p_cat = xr_cat * jnp.abs(ar_cat)
    qT_cat = xlT_cat * jnp.abs(ac_cat)
    ar2 = jnp.dot(xr_cat, ac_blk, preferred_element_type=jnp.float32)
    al2 = jnp.dot(ar_blk, xlT_cat, preferred_element_type=jnp.float32)
    sp, sv = _scores_pair(p_cat, qT_cat,
                          ar2[:, 0:1] + al2[0:1, :],
                          ar2[:, 1:2] + al2[1:2, :], coef_ref)
    ep, ip = _softmax_unnorm(sp)
    ev, iv = _softmax_unnorm(sv)
    op = jax.lax.dot_general(ep, xlT_cat[:_H, :], (((1,), (1,)), ((), ())),
                             preferred_element_type=jnp.float32) * ip
    ov = jax.lax.dot_general(ev, xlT_cat[_H:, :], (((1,), (1,)), ((), ())),
                             preferred_element_type=jnp.float32) * iv
    h_cat = jnp.concatenate([op, ov], axis=1) + b_cat
    return jnp.maximum(h_cat, 0.0)


def _net_kernel(x_ref,
                wl1_ref, wr1_ref, ar1_ref, ac1_ref, arb1_ref, acb1_ref,
                c1_ref, b1_ref,
                wl2_ref, wr2_ref, ar2_ref, ac2_ref, arb2_ref, acb2_ref,
                c2_ref, b2_ref, out_ref):
    x = x_ref[0]
    h1 = _dual_layer(x, wl1_ref[...], wr1_ref[...], ar1_ref[...],
                     ac1_ref[...], arb1_ref[...], acb1_ref[...],
                     c1_ref, b1_ref[...])
    h2 = _dual_layer(h1, wl2_ref[...], wr2_ref[...], ar2_ref[...],
                     ac2_ref[...], arb2_ref[...], acb2_ref[...],
                     c2_ref, b2_ref[...])
    mean = jnp.mean(h2, axis=0, keepdims=True)
    out_ref[0, 0] = mean[:, :_H]
    out_ref[0, 1] = mean[:, _H:]


def kernel(features, pi1_Wl, pi1_Wr, pi1_a, pi1_b, pi2_Wl, pi2_Wr, pi2_a, pi2_b,
           vf1_Wl, vf1_Wr, vf1_a, vf1_b, vf2_Wl, vf2_Wr, vf2_a, vf2_b):
    B = features.shape[0]
    H = _H
    z = jnp.zeros((H, H), jnp.float32)
    zv = jnp.zeros((H,), jnp.float32)

    wl1 = jnp.concatenate([pi1_Wl, vf1_Wl], axis=1)
    wr1 = jnp.concatenate([pi1_Wr, vf1_Wr], axis=1)
    wl2 = jnp.block([[pi2_Wl, z], [z, vf2_Wl]])
    wr2 = jnp.block([[pi2_Wr, z], [z, vf2_Wr]])

    a1c = jnp.concatenate([pi1_a, vf1_a])
    a2c = jnp.concatenate([pi2_a, vf2_a])
    ar1 = a1c[None, :]
    ar2 = a2c[None, :]
    ac1 = a1c[:, None]
    ac2 = a2c[:, None]
    arb1 = 0.2 * jnp.stack([jnp.concatenate([pi1_a, zv]),
                            jnp.concatenate([zv, vf1_a])])
    arb2 = 0.2 * jnp.stack([jnp.concatenate([pi2_a, zv]),
                            jnp.concatenate([zv, vf2_a])])
    acb1 = arb1.T
    acb2 = arb2.T
    c1 = (0.8 * jnp.where(a1c >= 0, 1.0, -1.0))[None, :]
    c2 = (0.8 * jnp.where(a2c >= 0, 1.0, -1.0))[None, :]
    b1 = jnp.concatenate([pi1_b, vf1_b])[None, :]
    b2 = jnp.concatenate([pi2_b, vf2_b])[None, :]

    full = lambda arr: pl.BlockSpec(arr.shape, lambda b: (0,) * arr.ndim)
    smem = lambda arr: pl.BlockSpec(arr.shape, lambda b: (0,) * arr.ndim,
                                    memory_space=pltpu.SMEM)

    out = pl.pallas_call(
        _net_kernel,
        grid=(B,),
        in_specs=[
            pl.BlockSpec((1, _N, _F), lambda b: (b, 0, 0)),
            full(wl1), full(wr1), full(ar1), full(ac1), full(arb1),
            full(acb1), smem(c1), full(b1),
            full(wl2), full(wr2), full(ar2), full(ac2), full(arb2),
            full(acb2), smem(c2), full(b2),
        ],
        out_specs=pl.BlockSpec((1, 2, 1, H), lambda b: (b, 0, 0, 0)),
        out_shape=jax.ShapeDtypeStruct((B, 2, 1, H), jnp.float32),
    )(features, wl1, wr1, ar1, ac1, arb1, acb1, c1, b1,
      wl2, wr2, ar2, ac2, arb2, acb2, c2, b2)

    return out[:, 0, 0, :], out[:, 1, 0, :]
